# baseline (speedup 1.0000x reference)
import functools

import jax
import jax.numpy as jnp
from jax import lax
from jax.experimental import pallas as pl
from jax.experimental.pallas import tpu as pltpu
from jax.experimental.pallas import tpu_sc as plsc

V = 100000
D = 64
K = 8
B = 16384

NC = 2
NS = 16
NW = NC * NS
L = 16

BPW = B // NW
CHUNK = 128
NCHUNK = BPW // CHUNK
GROUPS = CHUNK // L


def _splat(val, dtype=jnp.int32):
    return jnp.full((L,), val, dtype=dtype)


def _sense_kernel(word_hbm, ctx_hbm, tgt_hbm, wg_hbm, ws_hbm, out_hbm,
                  word_v, ctx_iv, tgt_iv, sense_v, ctxr_v, tgtr_v, out_v,
                  sem):
    wid = lax.axis_index("s") * NC + lax.axis_index("c")

    for chunk in range(NCHUNK):
        base = wid * BPW + chunk * CHUNK

        pltpu.sync_copy(word_hbm.at[pl.ds(base, CHUNK)], word_v)
        pltpu.sync_copy(ctx_hbm.at[pl.ds(base, CHUNK)], ctx_iv)
        pltpu.sync_copy(tgt_hbm.at[pl.ds(base, CHUNK)], tgt_iv)

        c1 = pltpu.async_copy(ws_hbm.at[word_v], sense_v, sem)
        c2 = pltpu.async_copy(wg_hbm.at[ctx_iv], ctxr_v, sem)
        c3 = pltpu.async_copy(wg_hbm.at[tgt_iv], tgtr_v, sem)
        c1.wait()
        c2.wait()
        c3.wait()

        def group_body(g, _):
            tok = g * L + lax.iota(jnp.int32, L)

            def score_body(d, accs):
                ctxv = plsc.load_gather(ctxr_v, [tok, _splat(d)])
                d8 = d * K
                new = []
                for k in range(K):
                    sv = plsc.load_gather(sense_v, [tok, _splat(d8 + k)])
                    new.append(accs[k] + ctxv * sv)
                return tuple(new)

            zeros = _splat(0.0, jnp.float32)
            accs = lax.fori_loop(0, D, score_body, (zeros,) * K, unroll=4)

            best = accs[0]
            bidx = _splat(0)
            for k in range(1, K):
                m = accs[k] > best
                best = jnp.where(m, accs[k], best)
                bidx = jnp.where(m, _splat(k), bidx)

            def dot_body(d, acc):
                chosen = plsc.load_gather(sense_v, [tok, _splat(d * K) + bidx])
                tv = plsc.load_gather(tgtr_v, [tok, _splat(d)])
                return acc + chosen * tv

            dot = lax.fori_loop(0, D, dot_body, zeros, unroll=8)
            res = 1.0 / (1.0 + jnp.exp(-dot))
            out_v[pl.ds(g * L, L)] = res
            return 0

        lax.fori_loop(0, GROUPS, group_body, 0)
        pltpu.sync_copy(out_v, out_hbm.at[pl.ds(base, CHUNK)])


@jax.jit
def _run(word, ctx, tgt, wg, ws2):
    mesh = plsc.VectorSubcoreMesh(core_axis_name="c", subcore_axis_name="s")
    f = functools.partial(
        pl.kernel,
        mesh=mesh,
        compiler_params=pltpu.CompilerParams(
            needs_layout_passes=False, use_tc_tiling_on_sc=False),
        out_type=jax.ShapeDtypeStruct((B,), jnp.float32),
        scratch_types=[
            pltpu.VMEM((CHUNK,), jnp.int32),
            pltpu.VMEM((CHUNK,), jnp.int32),
            pltpu.VMEM((CHUNK,), jnp.int32),
            pltpu.VMEM((CHUNK, D * K), jnp.float32),
            pltpu.VMEM((CHUNK, D), jnp.float32),
            pltpu.VMEM((CHUNK, D), jnp.float32),
            pltpu.VMEM((CHUNK,), jnp.float32),
            pltpu.SemaphoreType.DMA,
        ],
    )(_sense_kernel)
    return f(word, ctx, tgt, wg, ws2)


def kernel(x, W_g, W_s):
    word = x[0].astype(jnp.int32)
    ctx = x[1].astype(jnp.int32)
    tgt = x[2].astype(jnp.int32)
    ws2 = W_s.reshape(V, D * K)
    return _run(word, ctx, tgt, W_g, ws2)


# speedup vs baseline: 1.2010x; 1.2010x over previous
import functools

import jax
import jax.numpy as jnp
from jax import lax
from jax.experimental import pallas as pl
from jax.experimental.pallas import tpu as pltpu
from jax.experimental.pallas import tpu_sc as plsc

V = 100000
D = 64
K = 8
B = 16384

NC = 2
NS = 16
NW = NC * NS
L = 16

BPW = B // NW
CHUNK = 64
NCHUNK = BPW // CHUNK
GROUPS = CHUNK // L
NBUF = 2


def _splat(val, dtype=jnp.int32):
    return jnp.full((L,), val, dtype=dtype)


def _sense_kernel(word_hbm, ctxh_hbm, ctxo_hbm, tgth_hbm, tgto_hbm,
                  wg_hbm, ws_hbm, out_hbm, *scratch):
    word_v = scratch[0:2]
    ctxh_v = scratch[2:4]
    ctxo_v = scratch[4:6]
    tgth_v = scratch[6:8]
    tgto_v = scratch[8:10]
    sense_v = scratch[10:12]
    ctxr_v = scratch[12:14]
    tgtr_v = scratch[14:16]
    out_v = scratch[16:18]
    sems = scratch[18:20]
    osems = scratch[20:22]

    wid = lax.axis_index("s") * NC + lax.axis_index("c")
    base0 = wid * BPW

    def stage_and_fire(step, b):
        base = base0 + step * CHUNK
        pltpu.sync_copy(word_hbm.at[pl.ds(base, CHUNK)], word_v[b])
        pltpu.sync_copy(ctxh_hbm.at[pl.ds(base, CHUNK)], ctxh_v[b])
        pltpu.sync_copy(ctxo_hbm.at[pl.ds(base, CHUNK)], ctxo_v[b])
        pltpu.sync_copy(tgth_hbm.at[pl.ds(base, CHUNK)], tgth_v[b])
        pltpu.sync_copy(tgto_hbm.at[pl.ds(base, CHUNK)], tgto_v[b])
        c1 = pltpu.async_copy(ws_hbm.at[word_v[b]], sense_v[b], sems[b])
        c2 = pltpu.async_copy(wg_hbm.at[ctxh_v[b]], ctxr_v[b], sems[b])
        c3 = pltpu.async_copy(wg_hbm.at[tgth_v[b]], tgtr_v[b], sems[b])
        return (c1, c2, c3)

    def compute_chunk(b):
        def group_body(g, _):
            tok = g * L + lax.iota(jnp.int32, L)
            ctxoff = ctxo_v[b][pl.ds(g * L, L)]
            tgtoff = tgto_v[b][pl.ds(g * L, L)]

            def score_body(d, accs):
                ctxv = plsc.load_gather(ctxr_v[b], [tok, ctxoff + d])
                d8 = d * K
                new = []
                for k in range(K):
                    sv = plsc.load_gather(sense_v[b], [tok, _splat(d8 + k)])
                    new.append(accs[k] + ctxv * sv)
                return tuple(new)

            zeros = _splat(0.0, jnp.float32)
            accs = lax.fori_loop(0, D, score_body, (zeros,) * K, unroll=4)

            best = accs[0]
            bidx = _splat(0)
            for k in range(1, K):
                m = accs[k] > best
                best = jnp.where(m, accs[k], best)
                bidx = jnp.where(m, _splat(k), bidx)

            def dot_body(d, acc):
                chosen = plsc.load_gather(sense_v[b],
                                          [tok, _splat(d * K) + bidx])
                tv = plsc.load_gather(tgtr_v[b], [tok, tgtoff + d])
                return acc + chosen * tv

            dot = lax.fori_loop(0, D, dot_body, zeros, unroll=8)
            res = 1.0 / (1.0 + jnp.exp(-dot))
            out_v[b][pl.ds(g * L, L)] = res
            return 0

        lax.fori_loop(0, GROUPS, group_body, 0)

    copies = [stage_and_fire(0, 0), stage_and_fire(1, 1)]
    ocopies = [None, None]
    for step in range(NCHUNK):
        b = step % NBUF
        for c in copies[b]:
            c.wait()
        if ocopies[b] is not None:
            ocopies[b].wait()
        compute_chunk(b)
        oc = pltpu.async_copy(
            out_v[b], out_hbm.at[pl.ds(base0 + step * CHUNK, CHUNK)],
            osems[b])
        ocopies[b] = oc
        if step + NBUF < NCHUNK:
            copies[b] = stage_and_fire(step + NBUF, b)
    for b in range(NBUF):
        ocopies[b].wait()


@jax.jit
def _run(word, ctx_hi, ctx_off, tgt_hi, tgt_off, wg2, ws2):
    mesh = plsc.VectorSubcoreMesh(core_axis_name="c", subcore_axis_name="s")
    idx_t = pltpu.VMEM((CHUNK,), jnp.int32)
    f = functools.partial(
        pl.kernel,
        mesh=mesh,
        compiler_params=pltpu.CompilerParams(needs_layout_passes=False),
        out_type=jax.ShapeDtypeStruct((B,), jnp.float32),
        scratch_types=[idx_t] * (5 * NBUF) + [
            pltpu.VMEM((CHUNK, D * K), jnp.float32),
            pltpu.VMEM((CHUNK, D * K), jnp.float32),
            pltpu.VMEM((CHUNK, 2 * D), jnp.float32),
            pltpu.VMEM((CHUNK, 2 * D), jnp.float32),
            pltpu.VMEM((CHUNK, 2 * D), jnp.float32),
            pltpu.VMEM((CHUNK, 2 * D), jnp.float32),
            pltpu.VMEM((CHUNK,), jnp.float32),
            pltpu.VMEM((CHUNK,), jnp.float32),
            pltpu.SemaphoreType.DMA,
            pltpu.SemaphoreType.DMA,
            pltpu.SemaphoreType.DMA,
            pltpu.SemaphoreType.DMA,
        ],
    )(_sense_kernel)
    return f(word, ctx_hi, ctx_off, tgt_hi, tgt_off, wg2, ws2)


def kernel(x, W_g, W_s):
    word = x[0].astype(jnp.int32)
    ctx = x[1].astype(jnp.int32)
    tgt = x[2].astype(jnp.int32)
    ctx_hi = ctx >> 1
    ctx_off = (ctx & 1) * D
    tgt_hi = tgt >> 1
    tgt_off = (tgt & 1) * D
    wg2 = W_g.reshape(V // 2, 2 * D)
    ws2 = W_s.reshape(V, D * K)
    return _run(word, ctx_hi, ctx_off, tgt_hi, tgt_off, wg2, ws2)


# speedup vs baseline: 1.2429x; 1.0348x over previous
import functools

import jax
import jax.numpy as jnp
from jax import lax
from jax.experimental import pallas as pl
from jax.experimental.pallas import tpu as pltpu
from jax.experimental.pallas import tpu_sc as plsc

V = 100000
D = 64
K = 8
DK = D * K
B = 16384

NC = 2
NS = 16
NW = NC * NS
L = 16

BPW = B // NW
CHUNK = 64
NCHUNK = BPW // CHUNK
GROUPS = CHUNK // L
NBUF = 2


def _splat(val, dtype=jnp.int32):
    return jnp.full((L,), val, dtype=dtype)


def _sense_kernel(word_hbm, ctxh_hbm, ctxo_hbm, tgth_hbm, tgto_hbm,
                  wg_hbm, ws_hbm, out_hbm, *scratch):
    ctxo_v, tgto_v = scratch[0:2]
    word_v = scratch[2:4]
    ctxh_v = scratch[4:6]
    tgth_v = scratch[6:8]
    sense_v = scratch[8:10]
    ctxr_v = scratch[10:12]
    tgtr_v = scratch[12:14]
    out_v = scratch[14]
    sems = scratch[15:17]

    wid = lax.axis_index("s") * NC + lax.axis_index("c")
    base0 = wid * BPW

    pltpu.sync_copy(ctxo_hbm.at[pl.ds(base0, BPW)], ctxo_v)
    pltpu.sync_copy(tgto_hbm.at[pl.ds(base0, BPW)], tgto_v)

    def fire(step, b):
        base = base0 + step * CHUNK
        pltpu.sync_copy(word_hbm.at[pl.ds(base, CHUNK)], word_v[b])
        pltpu.sync_copy(ctxh_hbm.at[pl.ds(base, CHUNK)], ctxh_v[b])
        pltpu.sync_copy(tgth_hbm.at[pl.ds(base, CHUNK)], tgth_v[b])
        c1 = pltpu.async_copy(ws_hbm.at[word_v[b]], sense_v[b], sems[b])
        c2 = pltpu.async_copy(wg_hbm.at[ctxh_v[b]], ctxr_v[b], sems[b])
        c3 = pltpu.async_copy(wg_hbm.at[tgth_v[b]], tgtr_v[b], sems[b])
        return (c1, c2, c3)

    iota = lax.iota(jnp.int32, L)
    zv = _splat(0)
    zf = _splat(0.0, jnp.float32)

    def compute_chunk(step, b):
        def group_body(g, _):
            tok = g * L + iota
            s0 = step * CHUNK + g * L
            ctxoff = ctxo_v[pl.ds(s0, L)]
            tgtoff = tgto_v[pl.ds(s0, L)]

            def score_body(d, carry):
                accs = carry[0:K]
                scol, ccol = carry[K], carry[K + 1]
                ctxv = plsc.load_gather(ctxr_v[b], [tok, ccol])
                new = []
                for k in range(K):
                    sv = plsc.load_gather(sense_v[b], [tok, scol + k])
                    new.append(accs[k] + ctxv * sv)
                return tuple(new) + (scol + K, ccol + 1)

            init = (zf,) * K + (zv, ctxoff)
            res = lax.fori_loop(0, D, score_body, init, unroll=4)
            accs = res[0:K]

            best = accs[0]
            bidx = zv
            for k in range(1, K):
                m = accs[k] > best
                best = jnp.where(m, accs[k], best)
                bidx = jnp.where(m, _splat(k), bidx)

            def dot_body(d, carry):
                acc, dcol, tcol = carry
                chosen = plsc.load_gather(sense_v[b], [tok, dcol])
                tv = plsc.load_gather(tgtr_v[b], [tok, tcol])
                return (acc + chosen * tv, dcol + K, tcol + 1)

            dinit = (zf, bidx, tgtoff)
            dot, _, _ = lax.fori_loop(0, D, dot_body, dinit, unroll=8)
            res = 1.0 / (1.0 + jnp.exp(-dot))
            out_v[pl.ds(s0, L)] = res
            return 0

        lax.fori_loop(0, GROUPS, group_body, 0)

    copies = [fire(0, 0), fire(1, 1)]
    for step in range(NCHUNK):
        b = step % NBUF
        for c in copies[b]:
            c.wait()
        compute_chunk(step, b)
        if step + NBUF < NCHUNK:
            copies[b] = fire(step + NBUF, b)
    pltpu.sync_copy(out_v, out_hbm.at[pl.ds(base0, BPW)])


@jax.jit
def _run(word, ctx_hi, ctx_off, tgt_hi, tgt_off, wg2, ws2):
    mesh = plsc.VectorSubcoreMesh(core_axis_name="c", subcore_axis_name="s")
    idx_t = pltpu.VMEM((BPW,), jnp.int32)
    f = functools.partial(
        pl.kernel,
        mesh=mesh,
        compiler_params=pltpu.CompilerParams(needs_layout_passes=False),
        out_type=jax.ShapeDtypeStruct((B,), jnp.float32),
        scratch_types=[idx_t] * 2 + [pltpu.VMEM((CHUNK,), jnp.int32)] * 6 + [
            pltpu.VMEM((CHUNK, DK), jnp.float32),
            pltpu.VMEM((CHUNK, DK), jnp.float32),
            pltpu.VMEM((CHUNK, 2 * D), jnp.float32),
            pltpu.VMEM((CHUNK, 2 * D), jnp.float32),
            pltpu.VMEM((CHUNK, 2 * D), jnp.float32),
            pltpu.VMEM((CHUNK, 2 * D), jnp.float32),
            pltpu.VMEM((BPW,), jnp.float32),
            pltpu.SemaphoreType.DMA,
            pltpu.SemaphoreType.DMA,
        ],
    )(_sense_kernel)
    return f(word, ctx_hi, ctx_off, tgt_hi, tgt_off, wg2, ws2)


def kernel(x, W_g, W_s):
    word = x[0].astype(jnp.int32)
    ctx = x[1].astype(jnp.int32)
    tgt = x[2].astype(jnp.int32)
    ctx_hi = ctx >> 1
    ctx_off = (ctx & 1) * D
    tgt_hi = tgt >> 1
    tgt_off = (tgt & 1) * D
    wg2 = W_g.reshape(V // 2, 2 * D)
    ws2 = W_s.reshape(V, D * K)
    return _run(word, ctx_hi, ctx_off, tgt_hi, tgt_off, wg2, ws2)
